# baseline (speedup 1.0000x reference)
import functools

import jax
import jax.numpy as jnp
from jax import lax
from jax.experimental import pallas as pl
from jax.experimental.pallas import tpu as pltpu
from jax.experimental.pallas import tpu_sc as plsc

VOCAB = 10000
EMBED_DIM = 128
BATCH = 4096
HIST = 50

NUM_CORES = 2
NUM_SUBCORES = 16
LANES = 16
NW = NUM_CORES * NUM_SUBCORES
BPW = BATCH // NW
IPW = BPW * HIST
GROUPS = BPW // LANES

LUT_GRID = 2
LUT_BLK = 5120
LUT_PAD = LUT_GRID * LUT_BLK


SUBS = LUT_BLK // 128


def _lut_body(table_ref, w1_ref, w2_ref, b1_ref, b2_ref, s_ref):
    f32 = jnp.float32
    v = lax.dot_general(w2_ref[...], w1_ref[...], (((1,), (0,)), ((), ())),
                        precision=lax.Precision.HIGHEST,
                        preferred_element_type=f32)
    rows = [
        lax.dot_general(v, table_ref[pl.ds(p * 128, 128), :],
                        (((1,), (1,)), ((), ())),
                        precision=lax.Precision.DEFAULT,
                        preferred_element_type=f32)
        for p in range(SUBS)
    ]
    blk = jnp.concatenate(rows, axis=0)
    c = jnp.sum(b1_ref[...] * w2_ref[...]) + b2_ref[0, 0]
    i = pl.program_id(0)
    s_ref[pl.ds(i * SUBS, SUBS), :] = (blk + c) * (1.0 / HIST)


def _lut_call(table, w1, w2, b1r, b2r):
    return pl.pallas_call(
        _lut_body,
        grid=(LUT_GRID,),
        in_specs=[
            pl.BlockSpec((LUT_BLK, EMBED_DIM), lambda i: (i, 0)),
            pl.BlockSpec((EMBED_DIM, EMBED_DIM), lambda i: (0, 0)),
            pl.BlockSpec((1, EMBED_DIM), lambda i: (0, 0)),
            pl.BlockSpec((1, EMBED_DIM), lambda i: (0, 0)),
            pl.BlockSpec((1, 1), lambda i: (0, 0)),
        ],
        out_specs=pl.BlockSpec((LUT_GRID * SUBS, 128), lambda i: (0, 0)),
        out_shape=jax.ShapeDtypeStruct((LUT_GRID * SUBS, 128), jnp.float32),
    )(table, w1, w2, b1r, b2r)


@functools.cache
def _get_bag_kernel():
    mesh = plsc.VectorSubcoreMesh(core_axis_name="c", subcore_axis_name="s",
                                  num_cores=NUM_CORES,
                                  num_subcores=NUM_SUBCORES)

    @functools.partial(
        pl.kernel,
        out_type=jax.ShapeDtypeStruct((BATCH,), jnp.float32),
        mesh=mesh,
        scratch_types=[
            pltpu.VMEM((LUT_PAD,), jnp.float32),
            pltpu.VMEM((HIST, BPW), jnp.int32),
            pltpu.VMEM((BPW,), jnp.float32),
            pltpu.SemaphoreType.DMA,
            pltpu.SemaphoreType.DMA,
        ],
        compiler_params=pltpu.CompilerParams(needs_layout_passes=False,
                                             use_tc_tiling_on_sc=False),
    )
    def _bag_kernel(idx_hbm, lut_hbm, out_hbm, lut_v, idx_v, out_v,
                    sem_lut, sem_idx):
        wid = lax.axis_index("s") * NUM_CORES + lax.axis_index("c")
        cp_lut = pltpu.async_copy(lut_hbm, lut_v, sem_lut)
        cp_idx = pltpu.async_copy(idx_hbm.at[:, pl.ds(wid * BPW, BPW)],
                                  idx_v, sem_idx)
        cp_idx.wait()
        cp_lut.wait()

        def outer(g, _):
            def step(l, acc):
                iv = idx_v[l, pl.ds(g * LANES, LANES)]
                return acc + plsc.load_gather(lut_v, [iv])

            acc = lax.fori_loop(0, HIST, step, jnp.zeros((LANES,),
                                                         jnp.float32))
            out_v[pl.ds(g * LANES, LANES)] = acc
            return 0

        lax.fori_loop(0, GROUPS, outer, 0)
        pltpu.sync_copy(out_v, out_hbm.at[pl.ds(wid * BPW, BPW)])

    return _bag_kernel


def kernel(input, emb_table, W1, b1, W2, b2):
    lut = _lut_call(emb_table, W1, W2, b1.reshape(1, EMBED_DIM),
                    b2.reshape(1, 1))
    out = _get_bag_kernel()(input.T, lut.reshape(-1))
    return out.reshape(BATCH, 1)


# speedup vs baseline: 1.1297x; 1.1297x over previous
import functools

import jax
import jax.numpy as jnp
from jax import lax
from jax.experimental import pallas as pl
from jax.experimental.pallas import tpu as pltpu
from jax.experimental.pallas import tpu_sc as plsc

VOCAB = 10000
EMBED_DIM = 128
BATCH = 4096
HIST = 50

NUM_CORES = 2
NUM_SUBCORES = 16
LANES = 16
NW = NUM_CORES * NUM_SUBCORES
BPW = BATCH // NW
IPW = BPW * HIST
GROUPS = BPW // LANES

LUT_GRID = 2
LUT_BLK = 5120
LUT_PAD = LUT_GRID * LUT_BLK


SUBS = LUT_BLK // 128


def _lut_body(table_ref, w1_ref, w2_ref, b1_ref, b2_ref, int_ref, s_ref,
              idx_ref):
    f32 = jnp.float32
    v = lax.dot_general(w2_ref[...], w1_ref[...], (((1,), (0,)), ((), ())),
                        precision=lax.Precision.HIGHEST,
                        preferred_element_type=f32)
    rows = [
        lax.dot_general(v, table_ref[pl.ds(p * 128, 128), :],
                        (((1,), (1,)), ((), ())),
                        precision=lax.Precision.DEFAULT,
                        preferred_element_type=f32)
        for p in range(SUBS)
    ]
    blk = jnp.concatenate(rows, axis=0)
    c = jnp.sum(b1_ref[...] * w2_ref[...]) + b2_ref[0, 0]
    i = pl.program_id(0)
    s_ref[pl.ds(i * SUBS, SUBS), :] = (blk + c) * (1.0 / HIST)

    @pl.when(i == 0)
    def _():
        idx_ref[...] = int_ref[...].reshape(HIST, NW, BPW)


def _lut_call(table, w1, w2, b1r, b2r, inT):
    return pl.pallas_call(
        _lut_body,
        grid=(LUT_GRID,),
        in_specs=[
            pl.BlockSpec((LUT_BLK, EMBED_DIM), lambda i: (i, 0)),
            pl.BlockSpec((EMBED_DIM, EMBED_DIM), lambda i: (0, 0)),
            pl.BlockSpec((1, EMBED_DIM), lambda i: (0, 0)),
            pl.BlockSpec((1, EMBED_DIM), lambda i: (0, 0)),
            pl.BlockSpec((1, 1), lambda i: (0, 0)),
            pl.BlockSpec((HIST, BATCH), lambda i: (0, 0)),
        ],
        out_specs=[
            pl.BlockSpec((LUT_GRID * SUBS, 128), lambda i: (0, 0)),
            pl.BlockSpec((HIST, NW, BPW), lambda i: (0, 0, 0)),
        ],
        out_shape=[
            jax.ShapeDtypeStruct((LUT_GRID * SUBS, 128), jnp.float32),
            jax.ShapeDtypeStruct((HIST, NW, BPW), jnp.int32),
        ],
    )(table, w1, w2, b1r, b2r, inT)


@functools.cache
def _get_bag_kernel():
    mesh = plsc.VectorSubcoreMesh(core_axis_name="c", subcore_axis_name="s",
                                  num_cores=NUM_CORES,
                                  num_subcores=NUM_SUBCORES)

    @functools.partial(
        pl.kernel,
        out_type=jax.ShapeDtypeStruct((BATCH,), jnp.float32),
        mesh=mesh,
        scratch_types=[
            pltpu.VMEM((LUT_PAD,), jnp.float32),
            pltpu.VMEM((HIST, BPW), jnp.int32),
            pltpu.VMEM((BPW,), jnp.float32),
            pltpu.SemaphoreType.DMA,
            pltpu.SemaphoreType.DMA,
        ],
        compiler_params=pltpu.CompilerParams(needs_layout_passes=False,
                                             use_tc_tiling_on_sc=False),
    )
    def _bag_kernel(idx_hbm, lut_hbm, out_hbm, lut_v, idx_v, out_v,
                    sem_lut, sem_idx):
        wid = lax.axis_index("s") * NUM_CORES + lax.axis_index("c")
        cp_lut = pltpu.async_copy(lut_hbm, lut_v, sem_lut)
        cp_idx = pltpu.async_copy(idx_hbm.at[:, pl.ds(wid * BPW, BPW)],
                                  idx_v, sem_idx)
        cp_idx.wait()
        cp_lut.wait()

        def step(l, accs):
            new = []
            for g in range(GROUPS):
                iv = idx_v[l, pl.ds(g * LANES, LANES)]
                new.append(accs[g] + plsc.load_gather(lut_v, [iv]))
            return tuple(new)

        accs = lax.fori_loop(
            0, HIST, step,
            tuple(jnp.zeros((LANES,), jnp.float32) for _ in range(GROUPS)))
        for g in range(GROUPS):
            out_v[pl.ds(g * LANES, LANES)] = accs[g]
        pltpu.sync_copy(out_v, out_hbm.at[pl.ds(wid * BPW, BPW)])

    return _bag_kernel


def kernel(input, emb_table, W1, b1, W2, b2):
    lut, idx3 = _lut_call(emb_table, W1, W2, b1.reshape(1, EMBED_DIM),
                          b2.reshape(1, 1), input.T)
    out = _get_bag_kernel()(idx3.reshape(HIST, BATCH), lut.reshape(-1))
    return out.reshape(BATCH, 1)
